# baseline (speedup 1.0000x reference)
import jax
import jax.numpy as jnp
from jax.experimental import pallas as pl
from jax.experimental.pallas import tpu as pltpu

D = 2048
H = 256
L = 4
N = 4096
ROWS = 1024
NTILES = N // ROWS
CHUNK = 512
EPS = 1e-12


def _fused_kernel(ops_ref, w0_ref, w1_ref, w2_ref, w3_ref,
                  query_ref, qb_ref, ob_ref, ops_any,
                  qw0_any, qw1_any, qw2_any, qw3_any,
                  w1_any, w2_any, w3_any, out_ref,
                  e_s, wb_s, scores_s, row_s, proj_s, qw_stage, b_stage,
                  idx_s, sem, sem_qw, sem_b):
    j = pl.program_id(0)

    @pl.when(j == 0)
    def _():
        for l, wf in enumerate((w0_ref, w1_ref, w2_ref, w3_ref)):
            wb_s[l] = wf[:].astype(jnp.bfloat16)

    @pl.when(j < NTILES)
    def _():
        ops_tile = ops_ref[:].astype(jnp.bfloat16)
        for l in range(L):
            e_s[l, pl.ds(j * ROWS, ROWS), :] = jnp.dot(
                ops_tile, wb_s[l], preferred_element_type=jnp.float32)

    @pl.when(j == NTILES - 2)
    def _():
        pltpu.make_async_copy(qw0_any, qw_stage.at[0], sem_qw).start()

    @pl.when(j == NTILES - 1)
    def _():
        pltpu.make_async_copy(w1_any.at[pl.ds(D, D)], b_stage.at[0],
                              sem_b).start()

    @pl.when(j >= NTILES)
    def _():
        i = j - NTILES

        @pl.when(i == 0)
        def _():
            out_ref[:] = jnp.zeros_like(out_ref)

        qwbuf = qw_stage.at[jax.lax.rem(i, 2)]
        pltpu.make_async_copy(qw0_any, qwbuf, sem_qw).wait()
        qr = jnp.dot(query_ref[:], qwbuf[:],
                     preferred_element_type=jnp.float32) + qb_ref[
                         pl.ds(i, 1)][0]
        qnorm = jnp.maximum(jnp.sqrt(jnp.sum(qr * qr)), EPS)
        qn = qr / qnorm

        @pl.when(i > 0)
        def _():
            prev = idx_s[0]
            cp = pltpu.make_async_copy(
                ops_any.at[pl.ds(prev, 1)], row_s, sem)
            cp.start()
            cp.wait()
            bbuf = b_stage.at[jax.lax.rem(i - 1, 2)]
            pltpu.make_async_copy(
                w1_any.at[pl.ds(D, D)], bbuf, sem_b).wait()
            proj_s[:] = jnp.dot(row_s[:], bbuf[:],
                                preferred_element_type=jnp.float32)

        for l, qw_next in ((0, qw1_any), (1, qw2_any), (2, qw3_any)):
            @pl.when(i == l)
            def _(l=l, qw_next=qw_next):
                pltpu.make_async_copy(qw_next, qw_stage.at[(l + 1) % 2],
                                      sem_qw).start()
        for l, b_next in ((1, w2_any), (2, w3_any)):
            @pl.when(i == l)
            def _(l=l, b_next=b_next):
                pltpu.make_async_copy(
                    b_next.at[pl.ds(D, D)], b_stage.at[l % 2], sem_b).start()

        ob = ob_ref[pl.ds(i, 1)][0]
        c = jnp.where(i == 0, ob, proj_s[:] + ob)

        qc = jnp.sum(qn * c)
        cc = jnp.sum(c * c)

        for k in range(N // CHUNK):
            e = e_s[pl.ds(i, 1), pl.ds(k * CHUNK, CHUNK), :][0]
            s = jnp.sum(e * qn, axis=1)
            nn = jnp.sum(e * e, axis=1)
            dd = jnp.sum(e * c, axis=1)
            denom = jnp.maximum(
                jnp.sqrt(jnp.maximum(nn + 2.0 * dd + cc, 0.0)), EPS)
            sc = (s + qc) / denom
            scores_s[pl.ds(k * (CHUNK // 128), CHUNK // 128), :] = (
                sc.reshape(CHUNK // 128, 128))

        scores = scores_s[:]
        mx = jnp.max(scores)
        se = jnp.sum(jnp.exp(scores - mx))
        s00 = scores_s[0, 0]
        out_val = jnp.where(i == 0, s00 - mx, 0.0) - jnp.log(se)

        r_iota = jax.lax.broadcasted_iota(jnp.int32, (N // 128, 128), 0)
        l_iota = jax.lax.broadcasted_iota(jnp.int32, (N // 128, 128), 1)
        gidx = r_iota * 128 + l_iota
        amax = jnp.min(jnp.where(scores == mx, gidx, N))
        idx_s[0] = jnp.where(i == 0, 0, amax)

        o_iota = jax.lax.broadcasted_iota(jnp.int32, (8, 128), 0)
        z_iota = jax.lax.broadcasted_iota(jnp.int32, (8, 128), 1)
        mask = (o_iota == i) & (z_iota == 0)
        out_ref[:] = jnp.where(mask, out_val, out_ref[:])


@jax.jit
def kernel(query_embed, operators_embedding, params):
    ops = operators_embedding
    qb = jnp.stack([params['qb%d' % i] for i in range(L)])[:, None, :]
    ob = jnp.stack([params['ob%d' % i] for i in range(L)])[:, None, :]

    full = lambda shape: pl.BlockSpec(shape, lambda j: tuple(0 for _ in shape))
    anyspec = pl.BlockSpec(memory_space=pl.ANY)

    out_pad = pl.pallas_call(
        _fused_kernel,
        grid=(NTILES + L,),
        in_specs=[
            pl.BlockSpec((ROWS, D), lambda j: (jnp.minimum(j, NTILES - 1), 0)),
            full((D, H)),
            full((D, H)),
            full((D, H)),
            full((D, H)),
            full((1, D)),
            full((L, 1, H)),
            full((L, 1, H)),
        ] + [anyspec] * 8,
        out_specs=pl.BlockSpec((8, 128), lambda j: (0, 0)),
        out_shape=jax.ShapeDtypeStruct((8, 128), jnp.float32),
        scratch_shapes=[
            pltpu.VMEM((L, N, H), jnp.float32),
            pltpu.VMEM((L, D, H), jnp.bfloat16),
            pltpu.VMEM((N // 128, 128), jnp.float32),
            pltpu.VMEM((1, D), jnp.float32),
            pltpu.VMEM((1, H), jnp.float32),
            pltpu.VMEM((2, D, H), jnp.float32),
            pltpu.VMEM((2, D, H), jnp.float32),
            pltpu.SMEM((1,), jnp.int32),
            pltpu.SemaphoreType.DMA,
            pltpu.SemaphoreType.DMA,
            pltpu.SemaphoreType.DMA,
        ],
    )(ops, params['oW0'], params['oW1'], params['oW2'], params['oW3'],
      query_embed, qb, ob,
      ops, params['qW0'], params['qW1'], params['qW2'], params['qW3'],
      params['oW1'], params['oW2'], params['oW3'])

    return out_pad[:L, 0]


# speedup vs baseline: 1.0452x; 1.0452x over previous
import jax
import jax.numpy as jnp
from jax.experimental import pallas as pl
from jax.experimental.pallas import tpu as pltpu

D = 2048
H = 256
L = 4
N = 4096
ROWS = 512
NTILES = N // ROWS
CHUNK = 1024
EPS = 1e-12


def _fused_kernel(ops_ref, w0_ref, w1_ref, w2_ref, w3_ref,
                  query_ref, qb_ref, ob_ref, ops_any,
                  qw0_any, qw1_any, qw2_any, qw3_any,
                  w1_any, w2_any, w3_any, out_ref,
                  e_s, wb_s, scores_s, row_s, proj_s, qw_stage, b_stage,
                  idx_s, sem, sem_qw, sem_b):
    j = pl.program_id(0)

    @pl.when(j == 0)
    def _():
        for l, wf in enumerate((w0_ref, w1_ref, w2_ref, w3_ref)):
            wb_s[l] = wf[:].astype(jnp.bfloat16)

    @pl.when(j < NTILES)
    def _():
        ops_tile = ops_ref[:].astype(jnp.bfloat16)
        for l in range(L):
            e_s[l, pl.ds(j * ROWS, ROWS), :] = jnp.dot(
                ops_tile, wb_s[l], preferred_element_type=jnp.float32)

    @pl.when(j == NTILES - 2)
    def _():
        pltpu.make_async_copy(qw0_any, qw_stage.at[0], sem_qw).start()

    @pl.when(j == NTILES - 1)
    def _():
        pltpu.make_async_copy(w1_any.at[pl.ds(D, D)], b_stage.at[0],
                              sem_b).start()

    @pl.when(j >= NTILES)
    def _():
        i = j - NTILES

        @pl.when(i == 0)
        def _():
            out_ref[:] = jnp.zeros_like(out_ref)

        qwbuf = qw_stage.at[jax.lax.rem(i, 2)]
        pltpu.make_async_copy(qw0_any, qwbuf, sem_qw).wait()
        qr = jnp.dot(query_ref[:], qwbuf[:],
                     preferred_element_type=jnp.float32) + qb_ref[
                         pl.ds(i, 1)][0]
        qnorm = jnp.maximum(jnp.sqrt(jnp.sum(qr * qr)), EPS)
        qn = qr / qnorm

        @pl.when(i > 0)
        def _():
            prev = idx_s[0]
            cp = pltpu.make_async_copy(
                ops_any.at[pl.ds(prev, 1)], row_s, sem)
            cp.start()
            cp.wait()
            bbuf = b_stage.at[jax.lax.rem(i - 1, 2)]
            pltpu.make_async_copy(
                w1_any.at[pl.ds(D, D)], bbuf, sem_b).wait()
            proj_s[:] = jnp.dot(row_s[:], bbuf[:],
                                preferred_element_type=jnp.float32)

        for l, qw_next in ((0, qw1_any), (1, qw2_any), (2, qw3_any)):
            @pl.when(i == l)
            def _(l=l, qw_next=qw_next):
                pltpu.make_async_copy(qw_next, qw_stage.at[(l + 1) % 2],
                                      sem_qw).start()
        for l, b_next in ((1, w2_any), (2, w3_any)):
            @pl.when(i == l)
            def _(l=l, b_next=b_next):
                pltpu.make_async_copy(
                    b_next.at[pl.ds(D, D)], b_stage.at[l % 2], sem_b).start()

        ob = ob_ref[pl.ds(i, 1)][0]
        c = jnp.where(i == 0, ob, proj_s[:] + ob)

        qc = jnp.sum(qn * c)
        cc = jnp.sum(c * c)

        for k in range(N // CHUNK):
            e = e_s[pl.ds(i, 1), pl.ds(k * CHUNK, CHUNK), :][0]
            s = jnp.sum(e * qn, axis=1)
            nd = jnp.sum(e * (e + 2.0 * c), axis=1)
            denom = jnp.maximum(
                jnp.sqrt(jnp.maximum(nd + cc, 0.0)), EPS)
            sc = (s + qc) / denom
            scores_s[pl.ds(k * (CHUNK // 128), CHUNK // 128), :] = (
                sc.reshape(CHUNK // 128, 128))

        scores = scores_s[:]
        mx = jnp.max(scores)
        se = jnp.sum(jnp.exp(scores - mx))
        s00 = scores_s[0, 0]
        out_val = jnp.where(i == 0, s00 - mx, 0.0) - jnp.log(se)

        r_iota = jax.lax.broadcasted_iota(jnp.int32, (N // 128, 128), 0)
        l_iota = jax.lax.broadcasted_iota(jnp.int32, (N // 128, 128), 1)
        gidx = r_iota * 128 + l_iota
        amax = jnp.min(jnp.where(scores == mx, gidx, N))
        idx_s[0] = jnp.where(i == 0, 0, amax)

        o_iota = jax.lax.broadcasted_iota(jnp.int32, (8, 128), 0)
        z_iota = jax.lax.broadcasted_iota(jnp.int32, (8, 128), 1)
        mask = (o_iota == i) & (z_iota == 0)
        out_ref[:] = jnp.where(mask, out_val, out_ref[:])


@jax.jit
def kernel(query_embed, operators_embedding, params):
    ops = operators_embedding
    qb = jnp.stack([params['qb%d' % i] for i in range(L)])[:, None, :]
    ob = jnp.stack([params['ob%d' % i] for i in range(L)])[:, None, :]

    full = lambda shape: pl.BlockSpec(shape, lambda j: tuple(0 for _ in shape))
    anyspec = pl.BlockSpec(memory_space=pl.ANY)

    out_pad = pl.pallas_call(
        _fused_kernel,
        grid=(NTILES + L,),
        in_specs=[
            pl.BlockSpec((ROWS, D), lambda j: (jnp.minimum(j, NTILES - 1), 0)),
            full((D, H)),
            full((D, H)),
            full((D, H)),
            full((D, H)),
            full((1, D)),
            full((L, 1, H)),
            full((L, 1, H)),
        ] + [anyspec] * 8,
        out_specs=pl.BlockSpec((8, 128), lambda j: (0, 0)),
        out_shape=jax.ShapeDtypeStruct((8, 128), jnp.float32),
        scratch_shapes=[
            pltpu.VMEM((L, N, H), jnp.float32),
            pltpu.VMEM((L, D, H), jnp.bfloat16),
            pltpu.VMEM((N // 128, 128), jnp.float32),
            pltpu.VMEM((1, D), jnp.float32),
            pltpu.VMEM((1, H), jnp.float32),
            pltpu.VMEM((2, D, H), jnp.float32),
            pltpu.VMEM((2, D, H), jnp.float32),
            pltpu.SMEM((1,), jnp.int32),
            pltpu.SemaphoreType.DMA,
            pltpu.SemaphoreType.DMA,
            pltpu.SemaphoreType.DMA,
        ],
    )(ops, params['oW0'], params['oW1'], params['oW2'], params['oW3'],
      query_embed, qb, ob,
      ops, params['qW0'], params['qW1'], params['qW2'], params['qW3'],
      params['oW1'], params['oW2'], params['oW3'])

    return out_pad[:L, 0]
